# baseline (speedup 1.0000x reference)
import functools

import jax
import jax.numpy as jnp
from jax.experimental import pallas as pl
from jax.experimental.pallas import tpu as pltpu

_CHUNK_ROWS = 256


def _dma_pipe(emb_ref, out_ref, scratch, in_sems, out_sems, *, ch, nchunks):
    def in_copy(c):
        return pltpu.make_async_copy(
            emb_ref.at[pl.ds(c * ch, ch)], scratch.at[c], in_sems.at[c])

    def out_copy(c):
        return pltpu.make_async_copy(
            scratch.at[c], out_ref.at[pl.ds(c * ch, ch)], out_sems.at[c])

    for c in range(nchunks):
        in_copy(c).start()
    for c in range(nchunks):
        in_copy(c).wait()
        out_copy(c).start()
    for c in range(nchunks):
        out_copy(c).wait()


def kernel(inputs, embeddings):
    seq_len = inputs.shape[1]
    emb_dim = embeddings.shape[1]
    table = embeddings[:seq_len, :]
    ch = min(_CHUNK_ROWS, seq_len)
    nchunks = seq_len // ch
    out = pl.pallas_call(
        functools.partial(_dma_pipe, ch=ch, nchunks=nchunks),
        in_specs=[pl.BlockSpec(memory_space=pl.ANY)],
        out_specs=pl.BlockSpec(memory_space=pl.ANY),
        out_shape=jax.ShapeDtypeStruct((seq_len, emb_dim), embeddings.dtype),
        scratch_shapes=[
            pltpu.VMEM((nchunks, ch, emb_dim), embeddings.dtype),
            pltpu.SemaphoreType.DMA((nchunks,)),
            pltpu.SemaphoreType.DMA((nchunks,)),
        ],
    )(table)
    return out[None]


# speedup vs baseline: 1.0383x; 1.0383x over previous
import functools

import jax
import jax.numpy as jnp
from jax.experimental import pallas as pl
from jax.experimental.pallas import tpu as pltpu

_CHUNK_ROWS = 1024


def _dma_pipe(emb_ref, out_ref, scratch, in_sems, out_sems, *, ch, nchunks):
    def in_copy(c):
        return pltpu.make_async_copy(
            emb_ref.at[pl.ds(c * ch, ch)], scratch.at[c], in_sems.at[c])

    def out_copy(c):
        return pltpu.make_async_copy(
            scratch.at[c], out_ref.at[pl.ds(c * ch, ch)], out_sems.at[c])

    for c in range(nchunks):
        in_copy(c).start()
    for c in range(nchunks):
        in_copy(c).wait()
        out_copy(c).start()
    for c in range(nchunks):
        out_copy(c).wait()


def kernel(inputs, embeddings):
    seq_len = inputs.shape[1]
    emb_dim = embeddings.shape[1]
    table = embeddings[:seq_len, :]
    ch = min(_CHUNK_ROWS, seq_len)
    nchunks = seq_len // ch
    out = pl.pallas_call(
        functools.partial(_dma_pipe, ch=ch, nchunks=nchunks),
        in_specs=[pl.BlockSpec(memory_space=pl.ANY)],
        out_specs=pl.BlockSpec(memory_space=pl.ANY),
        out_shape=jax.ShapeDtypeStruct((seq_len, emb_dim), embeddings.dtype),
        scratch_shapes=[
            pltpu.VMEM((nchunks, ch, emb_dim), embeddings.dtype),
            pltpu.SemaphoreType.DMA((nchunks,)),
            pltpu.SemaphoreType.DMA((nchunks,)),
        ],
    )(table)
    return out[None]
